# baseline (speedup 1.0000x reference)
import functools

import jax
import jax.numpy as jnp
from jax import lax
from jax.experimental import pallas as pl
from jax.experimental.pallas import tpu as pltpu
from jax.experimental.pallas import tpu_sc as plsc

A0 = 0.5291772105638411
KE = 14.399645351950548

ABLATE_SCATTER = False
ABLATE_INPUT = False

NC = 2
NS = 16
LANES = 16
CHUNK = 1280
GRPS = CHUNK // LANES
ROWS = CHUNK // 128
YTAB = 2048
GTAB = 2048
GMAX = 5.0
KG = (GTAB - 1) / GMAX


@functools.lru_cache(maxsize=4)
def _build(N, E_pad):
    n_chunks = E_pad // CHUNK
    n_workers = NC * NS
    k_iters = -(-n_chunks // n_workers)
    full, rem = divmod(n_chunks, n_workers)
    t_half = (k_iters + 1) // 2
    S = ((-(-N // NS)) + 127) // 128 * 128
    acc_n = NS * S

    mesh = plsc.VectorSubcoreMesh(
        core_axis_name="c", subcore_axis_name="s",
        num_cores=NC, num_subcores=NS)

    edge_buf = [
        pltpu.VMEM((CHUNK,), jnp.int32),
        pltpu.VMEM((CHUNK,), jnp.int32),
        pltpu.VMEM((CHUNK,), jnp.float32),
        pltpu.VMEM((CHUNK,), jnp.float32),
        pltpu.VMEM((ROWS, 128), jnp.float32),
        pltpu.VMEM((ROWS, 128), jnp.int32),
        pltpu.SemaphoreType.DMA,
        pltpu.SemaphoreType.DMA,
    ]

    @functools.partial(
        pl.kernel,
        out_type=jax.ShapeDtypeStruct((NC, NS, S), jnp.float32),
        mesh=mesh,
        compiler_params=pltpu.CompilerParams(needs_layout_passes=False),
        scratch_types=[
            pltpu.VMEM((N,), jnp.int32),
            pltpu.VMEM((128,), jnp.float32),
            pltpu.VMEM((128,), jnp.float32),
            pltpu.VMEM((YTAB,), jnp.float32),
            pltpu.VMEM((GTAB,), jnp.float32),
            pltpu.VMEM_SHARED((acc_n,), jnp.float32),
        ] + edge_buf + edge_buf,
    )
    def zbl(anum_hbm, s_hbm, r_hbm, cut_hbm, len_hbm, par_hbm, lut_hbm,
            ytab_hbm, gtab_hbm, zeros_hbm, out_hbm,
            anum_v, lut_v, par_v, ytab_v, gtab_v, accum,
            s_v0, r_v0, cut_v0, len_v0, e_v0, ridx_v0, sem_in0, sem_sc0,
            s_v1, r_v1, cut_v1, len_v1, e_v1, ridx_v1, sem_in1, sem_sc1):
        cid = lax.axis_index("c")
        sid = lax.axis_index("s")
        wid = sid * NC + cid
        my_iters = full + jnp.where(wid < rem, 1, 0)

        bufs = ((s_v0, r_v0, cut_v0, len_v0, e_v0, ridx_v0, sem_in0, sem_sc0),
                (s_v1, r_v1, cut_v1, len_v1, e_v1, ridx_v1, sem_in1, sem_sc1))

        pltpu.sync_copy(anum_hbm, anum_v)
        pltpu.sync_copy(par_hbm, par_v)
        pltpu.sync_copy(lut_hbm, lut_v)
        pltpu.sync_copy(ytab_hbm, ytab_v)
        pltpu.sync_copy(gtab_hbm, gtab_v)

        base = sid * S
        pltpu.sync_copy(zeros_hbm, accum.at[pl.ds(base, S)])
        plsc.subcore_barrier()

        kyv = par_v[pl.ds(0, 16)]

        def in_descs(k, p):
            cbase = (wid + k * n_workers) * CHUNK
            s_v, r_v, cut_v, len_v, _, _, sem_in, _ = bufs[p]
            descs = [
                pltpu.make_async_copy(s_hbm.at[pl.ds(cbase, CHUNK)], s_v, sem_in),
                pltpu.make_async_copy(r_hbm.at[pl.ds(cbase, CHUNK)], r_v, sem_in),
                pltpu.make_async_copy(cut_hbm.at[pl.ds(cbase, CHUNK)], cut_v, sem_in),
                pltpu.make_async_copy(len_hbm.at[pl.ds(cbase, CHUNK)], len_v, sem_in),
            ]
            if ABLATE_INPUT:
                return descs[1:2]
            return descs

        def sc_descs(p):
            _, _, _, _, e_v, ridx_v, _, sem_sc = bufs[p]
            return [
                pltpu.make_async_copy(
                    e_v.at[r], accum.at[ridx_v.at[r]], sem_sc)
                for r in range(ROWS)
            ]

        def compute(p):
            s_v, r_v, cut_v, len_v, e_v, ridx_v, _, _ = bufs[p]

            def grp(i, carry):
                off = i * LANES
                sidx = s_v[pl.ds(off, LANES)]
                ridx = r_v[pl.ds(off, LANES)]
                izj = plsc.load_gather(anum_v, [sidx])
                izi = plsc.load_gather(anum_v, [ridx])
                ci = cut_v[pl.ds(off, LANES)]
                li = len_v[pl.ds(off, LANES)]
                pi = plsc.load_gather(lut_v, [izi])
                pj = plsc.load_gather(lut_v, [izj])
                zz = (izi * izj).astype(jnp.float32)
                rzd = li * (pi + pj)
                ty = jnp.minimum(rzd * kyv, YTAB - 1.001)
                ity = ty.astype(jnp.int32)
                fy = ty - ity.astype(jnp.float32)
                y0 = plsc.load_gather(ytab_v, [ity])
                y1 = plsc.load_gather(ytab_v, [ity + 1])
                y = y0 + fy * (y1 - y0)
                tg = jnp.minimum(li * KG, GTAB - 1.001)
                itg = tg.astype(jnp.int32)
                fg = tg - itg.astype(jnp.float32)
                g0 = plsc.load_gather(gtab_v, [itg])
                g1 = plsc.load_gather(gtab_v, [itg + 1])
                g = g0 + fg * (g1 - g0)
                e16 = ci * zz * y * g
                r = i // 8
                g = i % 8
                e_v[r, pl.ds(g * LANES, LANES)] = e16
                ridx_v[r, pl.ds(g * LANES, LANES)] = ridx
                return carry
            lax.fori_loop(0, GRPS, grp, 0, unroll=8)

        def step(k, p):
            @pl.when(k + 1 < my_iters)
            def _():
                for dsc in in_descs(k + 1, 1 - p):
                    dsc.start()

            @pl.when(k < my_iters)
            def _():
                for dsc in in_descs(k, p):
                    dsc.wait()

                if not ABLATE_SCATTER:
                    @pl.when(k >= 2)
                    def _():
                        for dsc in sc_descs(p):
                            dsc.wait()

                compute(p)
                if not ABLATE_SCATTER:
                    for dsc in sc_descs(p):
                        dsc.start(add=True)

        @pl.when(my_iters > 0)
        def _():
            for dsc in in_descs(0, 0):
                dsc.start()

        def pair(t, carry):
            step(2 * t, 0)
            step(2 * t + 1, 1)
            return carry
        lax.fori_loop(0, t_half, pair, 0)

        if not ABLATE_SCATTER:
            @pl.when(my_iters >= 2)
            def _():
                for p in (0, 1):
                    for dsc in sc_descs(p):
                        dsc.wait()

            @pl.when(my_iters == 1)
            def _():
                for dsc in sc_descs(0):
                    dsc.wait()

        plsc.subcore_barrier()
        pltpu.sync_copy(accum.at[pl.ds(base, S)], out_hbm.at[cid, sid])

    return zbl, S


def kernel(atomic_numbers, cutoffs, senders, receivers, lengths, num_nodes,
           a1_raw, a2_raw, a3_raw, a4_raw, c1_raw, c2_raw, c3_raw, c4_raw,
           p_raw, d_raw):
    N = atomic_numbers.shape[0]
    E = senders.shape[0]
    sp = jax.nn.softplus
    a1, a2, a3, a4 = sp(a1_raw), sp(a2_raw), sp(a3_raw), sp(a4_raw)
    c1, c2, c3, c4 = sp(c1_raw), sp(c2_raw), sp(c3_raw), sp(c4_raw)
    p = sp(p_raw)
    d = sp(d_raw)
    c_sum = c1 + c2 + c3 + c4
    scale = KE * 0.5 / c_sum
    lut = (d * jnp.power(jnp.arange(128, dtype=jnp.float32), p)
           ).astype(jnp.float32)

    rzd_max = 2.0 * GMAX * d * jnp.power(jnp.float32(93.0), p)
    ky = (YTAB - 1) / rzd_max
    xs_y = jnp.arange(YTAB, dtype=jnp.float32) * (rzd_max / (YTAB - 1))
    ytab = (scale * (c1 * jnp.exp(-a1 * xs_y) + c2 * jnp.exp(-a2 * xs_y)
                     + c3 * jnp.exp(-a3 * xs_y) + c4 * jnp.exp(-a4 * xs_y))
            ).astype(jnp.float32)

    def _sigma(x):
        return jnp.where(x > 0, jnp.exp(-1.0 / jnp.maximum(x, 1e-12)), 0.0)

    xs_g = jnp.arange(GTAB, dtype=jnp.float32) * (GMAX / (GTAB - 1))
    cc = xs_g / 1.5
    s1 = _sigma(1.0 - cc)
    s2 = _sigma(cc)
    w = s1 / (s1 + s2 + 1e-12)
    gtab = (w / jnp.maximum(xs_g, 1e-6)).astype(jnp.float32)

    params = jnp.broadcast_to(ky, (128,)).astype(jnp.float32)

    snd = senders.astype(jnp.int32)
    rcv = receivers.astype(jnp.int32)
    cut = cutoffs.reshape(E)
    lens = lengths.reshape(E)

    E_pad = -(-E // CHUNK) * CHUNK
    if E_pad != E:
        pad = E_pad - E
        snd = jnp.pad(snd, (0, pad))
        rcv = jnp.pad(rcv, (0, pad))
        cut = jnp.pad(cut, (0, pad))
        lens = jnp.pad(lens, (0, pad), constant_values=1.0)

    zbl, S = _build(N, E_pad)
    out = zbl(atomic_numbers.astype(jnp.int32), snd, rcv, cut, lens,
              params, lut, ytab, gtab, jnp.zeros((S,), jnp.float32))
    e_rep_node = out.reshape(NC, NS * S)[:, :N].sum(axis=0)
    return e_rep_node[:, None]


# speedup vs baseline: 2.9371x; 2.9371x over previous
import functools

import jax
import jax.numpy as jnp
from jax import lax
from jax.experimental import pallas as pl
from jax.experimental.pallas import tpu as pltpu
from jax.experimental.pallas import tpu_sc as plsc

A0 = 0.5291772105638411
KE = 14.399645351950548

ABLATE_SCATTER = False
ABLATE_INPUT = False

NC = 2
NS = 16
LANES = 16
CHUNK = 1280
GRPS = CHUNK // LANES
ROWS = CHUNK // 128
YTAB = 2048
GTAB = 2048
GMAX = 5.0
KG = (GTAB - 1) / GMAX


@functools.lru_cache(maxsize=4)
def _build(N, E_pad):
    n_chunks = E_pad // CHUNK
    n_workers = NC * NS
    k_iters = -(-n_chunks // n_workers)
    full, rem = divmod(n_chunks, n_workers)
    t_half = (k_iters + 1) // 2
    S = ((-(-N // NS)) + 127) // 128 * 128
    acc_n = NS * S

    mesh = plsc.VectorSubcoreMesh(
        core_axis_name="c", subcore_axis_name="s",
        num_cores=NC, num_subcores=NS)

    edge_buf = [
        pltpu.VMEM((CHUNK,), jnp.int32),
        pltpu.VMEM((CHUNK,), jnp.int32),
        pltpu.VMEM((CHUNK,), jnp.float32),
        pltpu.VMEM((CHUNK,), jnp.float32),
        pltpu.VMEM((ROWS, 128), jnp.float32),
        pltpu.VMEM((ROWS, 128), jnp.int32),
        pltpu.SemaphoreType.DMA,
        pltpu.SemaphoreType.DMA,
    ]

    @functools.partial(
        pl.kernel,
        out_type=jax.ShapeDtypeStruct((NC, NS, S), jnp.float32),
        mesh=mesh,
        compiler_params=pltpu.CompilerParams(needs_layout_passes=False),
        scratch_types=[
            pltpu.VMEM((N,), jnp.int32),
            pltpu.VMEM((128,), jnp.float32),
            pltpu.VMEM((128,), jnp.float32),
            pltpu.VMEM((YTAB,), jnp.float32),
            pltpu.VMEM((GTAB,), jnp.float32),
            pltpu.VMEM_SHARED((acc_n,), jnp.float32),
        ] + edge_buf + edge_buf,
    )
    def zbl(anum_hbm, s_hbm, r_hbm, cut_hbm, len_hbm, par_hbm, lut_hbm,
            ytab_hbm, gtab_hbm, zeros_hbm, out_hbm,
            anum_v, lut_v, par_v, ytab_v, gtab_v, accum,
            s_v0, r_v0, cut_v0, len_v0, e_v0, ridx_v0, sem_in0, sem_sc0,
            s_v1, r_v1, cut_v1, len_v1, e_v1, ridx_v1, sem_in1, sem_sc1):
        cid = lax.axis_index("c")
        sid = lax.axis_index("s")
        wid = sid * NC + cid
        my_iters = full + jnp.where(wid < rem, 1, 0)

        bufs = ((s_v0, r_v0, cut_v0, len_v0, e_v0, ridx_v0, sem_in0, sem_sc0),
                (s_v1, r_v1, cut_v1, len_v1, e_v1, ridx_v1, sem_in1, sem_sc1))

        pltpu.sync_copy(anum_hbm, anum_v)
        pltpu.sync_copy(par_hbm, par_v)
        pltpu.sync_copy(lut_hbm, lut_v)
        pltpu.sync_copy(ytab_hbm, ytab_v)
        pltpu.sync_copy(gtab_hbm, gtab_v)

        base = sid * S
        pltpu.sync_copy(zeros_hbm, accum.at[pl.ds(base, S)])
        plsc.subcore_barrier()

        kyv = par_v[pl.ds(0, 16)]

        def in_descs(k, p):
            cbase = (wid + k * n_workers) * CHUNK
            s_v, r_v, cut_v, len_v, _, _, sem_in, _ = bufs[p]
            descs = [
                pltpu.make_async_copy(s_hbm.at[pl.ds(cbase, CHUNK)], s_v, sem_in),
                pltpu.make_async_copy(r_hbm.at[pl.ds(cbase, CHUNK)], r_v, sem_in),
                pltpu.make_async_copy(cut_hbm.at[pl.ds(cbase, CHUNK)], cut_v, sem_in),
                pltpu.make_async_copy(len_hbm.at[pl.ds(cbase, CHUNK)], len_v, sem_in),
            ]
            if ABLATE_INPUT:
                return descs[1:2]
            return descs

        def sc_descs(p):
            _, _, _, _, e_v, ridx_v, _, sem_sc = bufs[p]
            return [
                pltpu.make_async_copy(
                    e_v.at[r], accum.at[ridx_v.at[r]], sem_sc)
                for r in range(ROWS)
            ]

        def compute(p):
            s_v, r_v, cut_v, len_v, e_v, ridx_v, _, _ = bufs[p]

            @plsc.parallel_loop(0, GRPS, unroll=4)
            def grp(i):
                off = i * LANES
                sidx = s_v[pl.ds(off, LANES)]
                ridx = r_v[pl.ds(off, LANES)]
                izj = plsc.load_gather(anum_v, [sidx])
                izi = plsc.load_gather(anum_v, [ridx])
                ci = cut_v[pl.ds(off, LANES)]
                li = len_v[pl.ds(off, LANES)]
                pi = plsc.load_gather(lut_v, [izi])
                pj = plsc.load_gather(lut_v, [izj])
                zz = (izi * izj).astype(jnp.float32)
                rzd = li * (pi + pj)
                ty = jnp.minimum(rzd * kyv, YTAB - 1.001)
                ity = ty.astype(jnp.int32)
                fy = ty - ity.astype(jnp.float32)
                y0 = plsc.load_gather(ytab_v, [ity])
                y1 = plsc.load_gather(ytab_v, [ity + 1])
                y = y0 + fy * (y1 - y0)
                tg = jnp.minimum(li * KG, GTAB - 1.001)
                itg = tg.astype(jnp.int32)
                fg = tg - itg.astype(jnp.float32)
                g0 = plsc.load_gather(gtab_v, [itg])
                g1 = plsc.load_gather(gtab_v, [itg + 1])
                g = g0 + fg * (g1 - g0)
                e16 = ci * zz * y * g
                r = i // 8
                g = i % 8
                e_v[r, pl.ds(g * LANES, LANES)] = e16
                ridx_v[r, pl.ds(g * LANES, LANES)] = ridx

        def step(k, p):
            @pl.when(k + 1 < my_iters)
            def _():
                for dsc in in_descs(k + 1, 1 - p):
                    dsc.start()

            @pl.when(k < my_iters)
            def _():
                for dsc in in_descs(k, p):
                    dsc.wait()

                if not ABLATE_SCATTER:
                    @pl.when(k >= 2)
                    def _():
                        for dsc in sc_descs(p):
                            dsc.wait()

                compute(p)
                if not ABLATE_SCATTER:
                    for dsc in sc_descs(p):
                        dsc.start(add=True)

        @pl.when(my_iters > 0)
        def _():
            for dsc in in_descs(0, 0):
                dsc.start()

        def pair(t, carry):
            step(2 * t, 0)
            step(2 * t + 1, 1)
            return carry
        lax.fori_loop(0, t_half, pair, 0)

        if not ABLATE_SCATTER:
            @pl.when(my_iters >= 2)
            def _():
                for p in (0, 1):
                    for dsc in sc_descs(p):
                        dsc.wait()

            @pl.when(my_iters == 1)
            def _():
                for dsc in sc_descs(0):
                    dsc.wait()

        plsc.subcore_barrier()
        pltpu.sync_copy(accum.at[pl.ds(base, S)], out_hbm.at[cid, sid])

    return zbl, S


def kernel(atomic_numbers, cutoffs, senders, receivers, lengths, num_nodes,
           a1_raw, a2_raw, a3_raw, a4_raw, c1_raw, c2_raw, c3_raw, c4_raw,
           p_raw, d_raw):
    N = atomic_numbers.shape[0]
    E = senders.shape[0]
    sp = jax.nn.softplus
    a1, a2, a3, a4 = sp(a1_raw), sp(a2_raw), sp(a3_raw), sp(a4_raw)
    c1, c2, c3, c4 = sp(c1_raw), sp(c2_raw), sp(c3_raw), sp(c4_raw)
    p = sp(p_raw)
    d = sp(d_raw)
    c_sum = c1 + c2 + c3 + c4
    scale = KE * 0.5 / c_sum
    lut = (d * jnp.power(jnp.arange(128, dtype=jnp.float32), p)
           ).astype(jnp.float32)

    rzd_max = 2.0 * GMAX * d * jnp.power(jnp.float32(93.0), p)
    ky = (YTAB - 1) / rzd_max
    xs_y = jnp.arange(YTAB, dtype=jnp.float32) * (rzd_max / (YTAB - 1))
    ytab = (scale * (c1 * jnp.exp(-a1 * xs_y) + c2 * jnp.exp(-a2 * xs_y)
                     + c3 * jnp.exp(-a3 * xs_y) + c4 * jnp.exp(-a4 * xs_y))
            ).astype(jnp.float32)

    def _sigma(x):
        return jnp.where(x > 0, jnp.exp(-1.0 / jnp.maximum(x, 1e-12)), 0.0)

    xs_g = jnp.arange(GTAB, dtype=jnp.float32) * (GMAX / (GTAB - 1))
    cc = xs_g / 1.5
    s1 = _sigma(1.0 - cc)
    s2 = _sigma(cc)
    w = s1 / (s1 + s2 + 1e-12)
    gtab = (w / jnp.maximum(xs_g, 1e-6)).astype(jnp.float32)

    params = jnp.broadcast_to(ky, (128,)).astype(jnp.float32)

    snd = senders.astype(jnp.int32)
    rcv = receivers.astype(jnp.int32)
    cut = cutoffs.reshape(E)
    lens = lengths.reshape(E)

    E_pad = -(-E // CHUNK) * CHUNK
    if E_pad != E:
        pad = E_pad - E
        snd = jnp.pad(snd, (0, pad))
        rcv = jnp.pad(rcv, (0, pad))
        cut = jnp.pad(cut, (0, pad))
        lens = jnp.pad(lens, (0, pad), constant_values=1.0)

    zbl, S = _build(N, E_pad)
    out = zbl(atomic_numbers.astype(jnp.int32), snd, rcv, cut, lens,
              params, lut, ytab, gtab, jnp.zeros((S,), jnp.float32))
    e_rep_node = out.reshape(NC, NS * S)[:, :N].sum(axis=0)
    return e_rep_node[:, None]
